# Initial kernel scaffold; baseline (speedup 1.0000x reference)
import jax
import jax.numpy as jnp
from jax.experimental import pallas as pl


def kernel(inp, shared_refpanel):
    raise NotImplementedError("write your pallas kernel here")


# speedup vs baseline: 5.1500x; 5.1500x over previous
import functools

import jax
import jax.numpy as jnp
from jax import lax
from jax.experimental import pallas as pl
from jax.experimental.pallas import tpu as pltpu
from jax.experimental.pallas import tpu_sc as plsc

ROWS = 64
N = 32768
TOPK = 64
L = 16
NVREG = N // L
CHUNKS = 64
VPC = NVREG // CHUNKS
CAP = 192
NW = 32
ROWS_PER_W = ROWS // NW

_NEG = float("-inf")
_BIGI = 0x7FFFFFFF

_GDN = lax.GatherDimensionNumbers(
    offset_dims=(), collapsed_slice_dims=(0,), start_index_map=(0,))


def _shuf(v, perm):
    return lax.gather(v, perm[:, None], _GDN, (1,),
                      mode=lax.GatherScatterMode.PROMISE_IN_BOUNDS)


def _bfly(v, op, lanes):
    for s in (8, 4, 2, 1):
        v = op(v, _shuf(v, lanes ^ s))
    return v


@functools.partial(
    pl.kernel,
    out_type=(
        jax.ShapeDtypeStruct((ROWS, TOPK), jnp.float32),
        jax.ShapeDtypeStruct((ROWS, TOPK), jnp.int32),
    ),
    mesh=plsc.VectorSubcoreMesh(core_axis_name="c", subcore_axis_name="s"),
    compiler_params=pltpu.CompilerParams(needs_layout_passes=False),
    scratch_types=[
        pltpu.VMEM((N,), jnp.float32),
        pltpu.VMEM((CAP * L,), jnp.float32),
        pltpu.VMEM((CAP * L,), jnp.int32),
        pltpu.VMEM((TOPK,), jnp.float32),
        pltpu.VMEM((TOPK,), jnp.int32),
    ],
)
def _topk_kernel(inp_hbm, vals_hbm, idxs_hbm, row_v, cval, cidx, oval, oidx):
    wid = lax.axis_index("c") * 16 + lax.axis_index("s")
    lanes = lax.iota(jnp.int32, L)
    neg = jnp.full((L,), _NEG, jnp.float32)
    bigi = jnp.full((L,), _BIGI, jnp.int32)

    for rr in range(ROWS_PER_W):
        r = rr * NW + wid
        pltpu.sync_copy(inp_hbm.at[r], row_v)

        def chunk_body(c, t0v):
            m = neg
            for j in range(VPC):
                m = jnp.maximum(m, row_v[pl.ds(c * (VPC * L) + j * L, L)])
            return jnp.minimum(t0v, _bfly(m, jnp.maximum, lanes))

        t0v = lax.fori_loop(
            0, CHUNKS, chunk_body, jnp.full((L,), float("inf"), jnp.float32))

        def clear_body(d, carry):
            cval[pl.ds(d * L, L)] = neg
            return carry

        lax.fori_loop(0, CAP, clear_body, 0)

        def collect(i, ptr):
            v = row_v[pl.ds(i * L, L)]
            msk = v >= t0v
            pos = jnp.minimum(ptr, CAP - 1) * L + lanes
            plsc.store_scatter(cval, [pos], v, mask=msk)
            plsc.store_scatter(cidx, [pos], i * L + lanes, mask=msk)
            return ptr + msk.astype(jnp.int32)

        ptr = lax.fori_loop(0, NVREG, collect, jnp.zeros((L,), jnp.int32))
        maxd = jnp.minimum(_bfly(ptr, jnp.maximum, lanes)[0], CAP)

        def round_body(k, carry):
            def scan_body(d, st):
                bv, bi, bd = st
                v = cval[pl.ds(d * L, L)]
                i = cidx[pl.ds(d * L, L)]
                better = (v > bv) | ((v == bv) & (i < bi))
                return (
                    jnp.where(better, v, bv),
                    jnp.where(better, i, bi),
                    jnp.where(better, d, bd),
                )

            bv, bi, bd = lax.fori_loop(
                0, maxd, scan_body,
                (neg, bigi, jnp.zeros((L,), jnp.int32)))
            mv = _bfly(bv, jnp.maximum, lanes)
            lm = bv == mv
            civ = _bfly(jnp.where(lm, bi, bigi), jnp.minimum, lanes)
            chosen = lm & (bi == civ)
            plsc.store_scatter(cval, [bd * L + lanes], neg, mask=chosen)
            l0 = lanes == 0
            kv = jnp.full((L,), k, jnp.int32)
            plsc.store_scatter(oval, [kv], mv, mask=l0)
            plsc.store_scatter(oidx, [kv], civ, mask=l0)
            return carry

        lax.fori_loop(0, TOPK, round_body, 0)

        pltpu.sync_copy(oval, vals_hbm.at[r])
        pltpu.sync_copy(oidx, idxs_hbm.at[r])


def kernel(inp, shared_refpanel):
    vals, idxs = _topk_kernel(inp)
    return (vals, idxs)
